# Initial kernel scaffold; baseline (speedup 1.0000x reference)
import jax
import jax.numpy as jnp
from jax.experimental import pallas as pl


def kernel(batch_node_tsr, edge_tsr_list, batch_last_node_idx_list, W_base0, b_base0, W_base1, b_base1, W_ad0, b_ad0, W_ad1, b_ad1):
    raise NotImplementedError("write your pallas kernel here")


# speedup vs baseline: 8.3605x; 8.3605x over previous
import functools

import jax
import jax.numpy as jnp
from jax import lax
from jax.experimental import pallas as pl
from jax.experimental.pallas import tpu as pltpu
from jax.experimental.pallas import tpu_sc as plsc

B, N, E, D = 4, 2500, 80000, 128
HALF = 2 * N
PAD = 5120
NP = 2 * PAD
NSC = 2
NTILE = 16
NW = NSC * NTILE
EPT = B * E // NW
CH = 125
NCH = EPT // CH
RPT = PAD // NTILE


def _make_sc_agg(ntab):
  outs = [jax.ShapeDtypeStruct((NP, D), jnp.float32) for _ in range(ntab)]
  scratch = [
      pltpu.VMEM((NCH, CH), jnp.int32),
      pltpu.VMEM((NCH, CH), jnp.int32),
      pltpu.VMEM((CH, D), jnp.float32),
  ]
  scratch += [pltpu.VMEM_SHARED((PAD, D), jnp.float32) for _ in range(ntab)]
  scratch.append(pltpu.SemaphoreType.DMA)
  mesh = plsc.VectorSubcoreMesh(core_axis_name="c", subcore_axis_name="s")

  def body(*refs):
    it = iter(refs)
    tabs = [next(it) for _ in range(ntab)]
    srcv_h = next(it)
    dstv_h = next(it)
    zero_h = next(it)
    outs_m = [next(it) for _ in range(ntab)]
    src_v = next(it)
    dst_v = next(it)
    rows_v = next(it)
    accs = [next(it) for _ in range(ntab)]
    sem = next(it)

    c = lax.axis_index("c")
    s = lax.axis_index("s")
    wid = c * NTILE + s
    base = s * RPT
    for a in accs:
      pltpu.sync_copy(zero_h, a.at[pl.ds(base, RPT)])
    pltpu.sync_copy(srcv_h.at[wid], src_v)
    pltpu.sync_copy(dstv_h.at[wid], dst_v)
    plsc.subcore_barrier()

    def chunk(j, carry):
      for t in range(ntab):
        pltpu.async_copy(tabs[t].at[src_v.at[j]], rows_v, sem).wait()
        pltpu.sync_copy(rows_v, accs[t].at[dst_v.at[j]], add=True)
      return carry

    lax.fori_loop(0, NCH, chunk, 0)
    plsc.subcore_barrier()
    orow = c * PAD + base
    for t in range(ntab):
      pltpu.sync_copy(accs[t].at[pl.ds(base, RPT)],
                      outs_m[t].at[pl.ds(orow, RPT)])

  return pl.kernel(body, out_type=tuple(outs), mesh=mesh,
                   scratch_types=scratch)


_sc_agg1 = _make_sc_agg(1)
_sc_agg2 = _make_sc_agg(2)

DEG_ROWS = PAD // D
EVR = EPT // 16


def _sc_deg_body(dst16_h, iota_h, zero_h, degout_h,
                 dst_v, idx_v, hist_v, deg_sh):
  c = lax.axis_index("c")
  s = lax.axis_index("s")
  wid = c * NTILE + s
  pltpu.sync_copy(dst16_h.at[wid], dst_v)
  pltpu.sync_copy(iota_h, idx_v)

  @pl.when(s == 0)
  def _():
    pltpu.sync_copy(zero_h, deg_sh)

  zero16 = jnp.zeros((16,), jnp.float32)

  def zrow(r, carry):
    for k in range(D // 16):
      hist_v[r, pl.ds(k * 16, 16)] = zero16
    return carry

  lax.fori_loop(0, DEG_ROWS, zrow, 0)
  ones16 = jnp.ones((16,), jnp.float32)

  def ebody(e, carry):
    iv = dst_v[e, :]
    r = lax.shift_right_logical(iv, 7)
    col = lax.bitwise_and(iv, 127)
    plsc.addupdate_scatter(hist_v, [r, col], ones16)
    return carry

  lax.fori_loop(0, EVR, ebody, 0)
  plsc.subcore_barrier()
  pltpu.sync_copy(hist_v, deg_sh.at[idx_v.at[0]], add=True)
  plsc.subcore_barrier()

  @pl.when(s == 0)
  def _():
    pltpu.sync_copy(deg_sh, degout_h.at[c])


_sc_deg = pl.kernel(
    _sc_deg_body,
    out_type=(jax.ShapeDtypeStruct((NSC, DEG_ROWS, D), jnp.float32),),
    mesh=plsc.VectorSubcoreMesh(core_axis_name="c", subcore_axis_name="s"),
    scratch_types=[
        pltpu.VMEM((EVR, 16), jnp.int32),
        pltpu.VMEM((1, DEG_ROWS), jnp.int32),
        pltpu.VMEM((DEG_ROWS, D), jnp.float32),
        pltpu.VMEM_SHARED((DEG_ROWS, D), jnp.float32),
    ],
    compiler_params=pltpu.CompilerParams(needs_layout_passes=False))

_TC_BLK = 1024


def _row_spec():
  return pl.BlockSpec((_TC_BLK, D), lambda i: (i, 0))


def _full_spec(shape):
  return pl.BlockSpec(shape, lambda i: (0, 0))


def _deg_spec():
  return pl.BlockSpec((_TC_BLK, 8), lambda i: (i, 0))


def _tc1_body(m_ref, d_ref, w_ref, b_ref, o_ref):
  x = m_ref[...] / jnp.maximum(d_ref[...][:, 0:1], 1.0)
  y = jnp.dot(x, w_ref[...], preferred_element_type=jnp.float32) + b_ref[...]
  o_ref[...] = jnp.maximum(y, 0.0)


def _tc1(m, deg, w, b):
  return pl.pallas_call(
      _tc1_body,
      grid=(NP // _TC_BLK,),
      in_specs=[_row_spec(), _deg_spec(), _full_spec((D, D)),
                _full_spec((1, D))],
      out_specs=_row_spec(),
      out_shape=jax.ShapeDtypeStruct((NP, D), jnp.float32),
  )(m, deg, w, b)


def _tcB_body(m1_ref, m0_ref, d_ref, wh_ref, wa_ref, wb_ref, bh_ref, bc_ref,
              h_ref, c_ref):
  dinv = 1.0 / jnp.maximum(d_ref[...][:, 0:1], 1.0)
  x1 = m1_ref[...] * dinv
  x0 = m0_ref[...] * dinv
  h = jnp.dot(x1, wh_ref[...], preferred_element_type=jnp.float32) + bh_ref[...]
  h_ref[...] = jnp.maximum(h, 0.0)
  cc = (jnp.dot(x1, wa_ref[...], preferred_element_type=jnp.float32)
        + jnp.dot(x0, wb_ref[...], preferred_element_type=jnp.float32)
        + bc_ref[...])
  c_ref[...] = jnp.maximum(cc, 0.0)


def _tcB(m1, m0, deg, wh, wa, wb, bh, bc):
  return pl.pallas_call(
      _tcB_body,
      grid=(NP // _TC_BLK,),
      in_specs=[_row_spec(), _row_spec(), _deg_spec(),
                _full_spec((D, D)), _full_spec((D, D)), _full_spec((D, D)),
                _full_spec((1, D)), _full_spec((1, D))],
      out_specs=[_row_spec(), _row_spec()],
      out_shape=[jax.ShapeDtypeStruct((NP, D), jnp.float32),
                 jax.ShapeDtypeStruct((NP, D), jnp.float32)],
  )(m1, m0, deg, wh, wa, wb, bh, bc)


def _tc2_body(ma_ref, mb_ref, d_ref, wa_ref, wb_ref, bc_ref, o_ref):
  dinv = 1.0 / jnp.maximum(d_ref[...][:, 0:1], 1.0)
  y = (jnp.dot(ma_ref[...] * dinv, wa_ref[...],
               preferred_element_type=jnp.float32)
       + jnp.dot(mb_ref[...] * dinv, wb_ref[...],
                 preferred_element_type=jnp.float32)
       + bc_ref[...])
  o_ref[...] = jnp.maximum(y, 0.0)


def _tc2(ma, mb, deg, wa, wb, bc):
  return pl.pallas_call(
      _tc2_body,
      grid=(NP // _TC_BLK,),
      in_specs=[_row_spec(), _row_spec(), _deg_spec(),
                _full_spec((D, D)), _full_spec((D, D)), _full_spec((1, D))],
      out_specs=_row_spec(),
      out_shape=jax.ShapeDtypeStruct((NP, D), jnp.float32),
  )(ma, mb, deg, wa, wb, bc)


def kernel(batch_node_tsr, edge_tsr_list, batch_last_node_idx_list,
           W_base0, b_base0, W_base1, b_base1,
           W_ad0, b_ad0, W_ad1, b_ad1):
  f32 = jnp.float32
  x = batch_node_tsr.reshape(B * N, D)
  xp = jnp.pad(x.reshape(NSC, HALF, D),
               ((0, 0), (0, PAD - HALF), (0, 0))).reshape(NP, D)

  src = edge_tsr_list[:, 0, :]
  dst = edge_tsr_list[:, 1, :]
  g = jnp.arange(B, dtype=jnp.int32)
  dst_local = dst + ((g % 2) * N)[:, None]
  src_p = (src + ((g // 2) * PAD + (g % 2) * N)[:, None]).reshape(NW, NCH, CH)
  dst_l = dst_local.reshape(NW, NCH, CH)
  dst16 = dst_local.reshape(NW, EVR, 16)

  zero_h = jnp.zeros((RPT, D), f32)
  zero_deg_h = jnp.zeros((DEG_ROWS, D), f32)
  iota_h = jnp.arange(DEG_ROWS, dtype=jnp.int32).reshape(1, DEG_ROWS)

  bb0 = b_base0.reshape(1, D)
  bb1 = b_base1.reshape(1, D)
  ba0 = b_ad0.reshape(1, D)
  ba1 = b_ad1.reshape(1, D)

  (deg_pck,) = _sc_deg(dst16, iota_h, zero_deg_h)
  deg8 = jnp.broadcast_to(deg_pck.reshape(NP, 1), (NP, 8))
  (m0,) = _sc_agg1(xp, src_p, dst_l, zero_h)
  h1 = _tc1(m0, deg8, W_base0, bb0)
  (m1,) = _sc_agg1(h1, src_p, dst_l, zero_h)
  h2, c1 = _tcB(m1, m0, deg8, W_base1, W_ad0[:D], W_ad0[D:], bb1, ba0)
  m2, m3 = _sc_agg2(h2, c1, src_p, dst_l, zero_h)
  c2 = _tc2(m2, m3, deg8, W_ad1[:D], W_ad1[D:], ba1)

  def unpad(y):
    return jnp.concatenate([y[0:HALF], y[PAD:PAD + HALF]],
                           axis=0).reshape(B, N, D)

  return unpad(h2), unpad(c2)


# speedup vs baseline: 10.6480x; 1.2736x over previous
import functools

import jax
import jax.numpy as jnp
from jax import lax
from jax.experimental import pallas as pl
from jax.experimental.pallas import tpu as pltpu
from jax.experimental.pallas import tpu_sc as plsc

B, N, E, D = 4, 2500, 80000, 128
HALF = 2 * N
PAD = 5120
NP = 2 * PAD
NSC = 2
NTILE = 16
NW = NSC * NTILE
EPT = B * E // NW
RPT = PAD // NTILE
CH1 = 125


def _make_sc_agg(ntab):
  ch = CH1
  nch = EPT // ch
  outs = [jax.ShapeDtypeStruct((NP, D), jnp.float32) for _ in range(ntab)]
  scratch = [
      pltpu.VMEM((nch, ch), jnp.int32),
      pltpu.VMEM((nch, ch), jnp.int32),
      pltpu.VMEM((ch, D), jnp.float32),
      pltpu.VMEM((ch, D), jnp.float32),
  ]
  scratch += [pltpu.VMEM_SHARED((PAD, D), jnp.float32) for _ in range(ntab)]
  scratch += [pltpu.SemaphoreType.DMA, pltpu.SemaphoreType.DMA]
  mesh = plsc.VectorSubcoreMesh(core_axis_name="c", subcore_axis_name="s")

  def body(*refs):
    it = iter(refs)
    tabs = [next(it) for _ in range(ntab)]
    srcv_h = next(it)
    dstv_h = next(it)
    zero_h = next(it)
    outs_m = [next(it) for _ in range(ntab)]
    src_v = next(it)
    dst_v = next(it)
    bufs = [next(it), next(it)]
    accs = [next(it) for _ in range(ntab)]
    sems = [next(it), next(it)]

    c = lax.axis_index("c")
    s = lax.axis_index("s")
    wid = c * NTILE + s
    base = s * RPT
    for a in accs:
      pltpu.sync_copy(zero_h, a.at[pl.ds(base, RPT)])
    pltpu.sync_copy(srcv_h.at[wid], src_v)
    pltpu.sync_copy(dstv_h.at[wid], dst_v)
    plsc.subcore_barrier()

    def gather(t, j, p):
      pltpu.async_copy(tabs[t].at[src_v.at[j]], bufs[p], sems[p])

    def wait(p):
      pltpu.make_async_copy(tabs[0].at[src_v.at[0]], bufs[p], sems[p]).wait()

    def scat(t, j, p):
      pltpu.sync_copy(bufs[p], accs[t].at[dst_v.at[j]], add=True)

    if ntab == 1:
      gather(0, 0, 0)

      def step(jj, carry):
        j0 = jj * 2
        wait(0)
        gather(0, j0 + 1, 1)
        scat(0, j0, 0)
        wait(1)
        gather(0, j0 + 2, 0)
        scat(0, j0 + 1, 1)
        return carry

      lax.fori_loop(0, nch // 2 - 1, step, 0)
      wait(0)
      gather(0, nch - 1, 1)
      scat(0, nch - 2, 0)
      wait(1)
      scat(0, nch - 1, 1)
    else:
      gather(0, 0, 0)

      def step(j, carry):
        wait(0)
        gather(1, j, 1)
        scat(0, j, 0)
        wait(1)
        gather(0, j + 1, 0)
        scat(1, j, 1)
        return carry

      lax.fori_loop(0, nch - 1, step, 0)
      wait(0)
      gather(1, nch - 1, 1)
      scat(0, nch - 1, 0)
      wait(1)
      scat(1, nch - 1, 1)

    plsc.subcore_barrier()
    orow = c * PAD + base
    for t in range(ntab):
      pltpu.sync_copy(accs[t].at[pl.ds(base, RPT)],
                      outs_m[t].at[pl.ds(orow, RPT)])

  return pl.kernel(body, out_type=tuple(outs), mesh=mesh,
                   scratch_types=scratch)


_sc_agg1 = _make_sc_agg(1)

DEG_ROWS = PAD // D
EVR = EPT // 16


def _sc_deg_body(dst16_h, iota_h, zero_h, degout_h,
                 dst_v, idx_v, hist_v, deg_sh):
  c = lax.axis_index("c")
  s = lax.axis_index("s")
  wid = c * NTILE + s
  pltpu.sync_copy(dst16_h.at[wid], dst_v)
  pltpu.sync_copy(iota_h, idx_v)

  @pl.when(s == 0)
  def _():
    pltpu.sync_copy(zero_h, deg_sh)

  zero16 = jnp.zeros((16,), jnp.float32)

  def zrow(r, carry):
    for k in range(D // 16):
      hist_v[r, pl.ds(k * 16, 16)] = zero16
    return carry

  lax.fori_loop(0, DEG_ROWS, zrow, 0)
  ones16 = jnp.ones((16,), jnp.float32)

  def ebody(e, carry):
    iv = dst_v[e, :]
    r = lax.shift_right_logical(iv, 7)
    col = lax.bitwise_and(iv, 127)
    plsc.addupdate_scatter(hist_v, [r, col], ones16)
    return carry

  lax.fori_loop(0, EVR, ebody, 0)
  plsc.subcore_barrier()
  pltpu.sync_copy(hist_v, deg_sh.at[idx_v.at[0]], add=True)
  plsc.subcore_barrier()

  @pl.when(s == 0)
  def _():
    pltpu.sync_copy(deg_sh, degout_h.at[c])


_sc_deg = pl.kernel(
    _sc_deg_body,
    out_type=(jax.ShapeDtypeStruct((NSC, DEG_ROWS, D), jnp.float32),),
    mesh=plsc.VectorSubcoreMesh(core_axis_name="c", subcore_axis_name="s"),
    scratch_types=[
        pltpu.VMEM((EVR, 16), jnp.int32),
        pltpu.VMEM((1, DEG_ROWS), jnp.int32),
        pltpu.VMEM((DEG_ROWS, D), jnp.float32),
        pltpu.VMEM_SHARED((DEG_ROWS, D), jnp.float32),
    ],
    compiler_params=pltpu.CompilerParams(needs_layout_passes=False))

_TC_BLK = 1024


def _row_spec():
  return pl.BlockSpec((_TC_BLK, D), lambda i: (i, 0))


def _full_spec(shape):
  return pl.BlockSpec(shape, lambda i: (0, 0))


def _deg_spec():
  return pl.BlockSpec((_TC_BLK, 8), lambda i: (i, 0))


def _tc1_body(m_ref, d_ref, w_ref, b_ref, o_ref):
  x = m_ref[...] / jnp.maximum(d_ref[...][:, 0:1], 1.0)
  y = jnp.dot(x, w_ref[...], preferred_element_type=jnp.float32) + b_ref[...]
  o_ref[...] = jnp.maximum(y, 0.0)


def _tc1(m, deg, w, b):
  return pl.pallas_call(
      _tc1_body,
      grid=(NP // _TC_BLK,),
      in_specs=[_row_spec(), _deg_spec(), _full_spec((D, D)),
                _full_spec((1, D))],
      out_specs=_row_spec(),
      out_shape=jax.ShapeDtypeStruct((NP, D), jnp.float32),
  )(m, deg, w, b)


def _tcB_body(m1_ref, m0_ref, d_ref, wh_ref, wa_ref, wb_ref, bh_ref, bc_ref,
              h_ref, c_ref):
  dinv = 1.0 / jnp.maximum(d_ref[...][:, 0:1], 1.0)
  x1 = m1_ref[...] * dinv
  x0 = m0_ref[...] * dinv
  h = jnp.dot(x1, wh_ref[...], preferred_element_type=jnp.float32) + bh_ref[...]
  h_ref[...] = jnp.maximum(h, 0.0)
  cc = (jnp.dot(x1, wa_ref[...], preferred_element_type=jnp.float32)
        + jnp.dot(x0, wb_ref[...], preferred_element_type=jnp.float32)
        + bc_ref[...])
  c_ref[...] = jnp.maximum(cc, 0.0)


def _tcB(m1, m0, deg, wh, wa, wb, bh, bc):
  return pl.pallas_call(
      _tcB_body,
      grid=(NP // _TC_BLK,),
      in_specs=[_row_spec(), _row_spec(), _deg_spec(),
                _full_spec((D, D)), _full_spec((D, D)), _full_spec((D, D)),
                _full_spec((1, D)), _full_spec((1, D))],
      out_specs=[_row_spec(), _row_spec()],
      out_shape=[jax.ShapeDtypeStruct((NP, D), jnp.float32),
                 jax.ShapeDtypeStruct((NP, D), jnp.float32)],
  )(m1, m0, deg, wh, wa, wb, bh, bc)


def _tc2_body(ma_ref, mb_ref, d_ref, wa_ref, wb_ref, bc_ref, o_ref):
  dinv = 1.0 / jnp.maximum(d_ref[...][:, 0:1], 1.0)
  y = (jnp.dot(ma_ref[...] * dinv, wa_ref[...],
               preferred_element_type=jnp.float32)
       + jnp.dot(mb_ref[...] * dinv, wb_ref[...],
                 preferred_element_type=jnp.float32)
       + bc_ref[...])
  o_ref[...] = jnp.maximum(y, 0.0)


def _tc2(ma, mb, deg, wa, wb, bc):
  return pl.pallas_call(
      _tc2_body,
      grid=(NP // _TC_BLK,),
      in_specs=[_row_spec(), _row_spec(), _deg_spec(),
                _full_spec((D, D)), _full_spec((D, D)), _full_spec((1, D))],
      out_specs=_row_spec(),
      out_shape=jax.ShapeDtypeStruct((NP, D), jnp.float32),
  )(ma, mb, deg, wa, wb, bc)


def kernel(batch_node_tsr, edge_tsr_list, batch_last_node_idx_list,
           W_base0, b_base0, W_base1, b_base1,
           W_ad0, b_ad0, W_ad1, b_ad1):
  f32 = jnp.float32
  x = batch_node_tsr.reshape(B * N, D)
  xp = jnp.pad(x.reshape(NSC, HALF, D),
               ((0, 0), (0, PAD - HALF), (0, 0))).reshape(NP, D)

  src = edge_tsr_list[:, 0, :]
  dst = edge_tsr_list[:, 1, :]
  g = jnp.arange(B, dtype=jnp.int32)
  dst_local = dst + ((g % 2) * N)[:, None]
  src_pad = src + ((g // 2) * PAD + (g % 2) * N)[:, None]
  src_p1 = src_pad.reshape(NW, EPT // CH1, CH1)
  dst_l1 = dst_local.reshape(NW, EPT // CH1, CH1)
  dst16 = dst_local.reshape(NW, EVR, 16)

  zero_h = jnp.zeros((RPT, D), f32)
  zero_deg_h = jnp.zeros((DEG_ROWS, D), f32)
  iota_h = jnp.arange(DEG_ROWS, dtype=jnp.int32).reshape(1, DEG_ROWS)

  bb0 = b_base0.reshape(1, D)
  bb1 = b_base1.reshape(1, D)
  ba0 = b_ad0.reshape(1, D)
  ba1 = b_ad1.reshape(1, D)

  (deg_pck,) = _sc_deg(dst16, iota_h, zero_deg_h)
  deg8 = jnp.broadcast_to(deg_pck.reshape(NP, 1), (NP, 8))
  (m0,) = _sc_agg1(xp, src_p1, dst_l1, zero_h)
  h1 = _tc1(m0, deg8, W_base0, bb0)
  (m1,) = _sc_agg1(h1, src_p1, dst_l1, zero_h)
  h2, c1 = _tcB(m1, m0, deg8, W_base1, W_ad0[:D], W_ad0[D:], bb1, ba0)
  (m2,) = _sc_agg1(h2, src_p1, dst_l1, zero_h)
  (m3,) = _sc_agg1(c1, src_p1, dst_l1, zero_h)
  c2 = _tc2(m2, m3, deg8, W_ad1[:D], W_ad1[D:], ba1)

  def unpad(y):
    return jnp.concatenate([y[0:HALF], y[PAD:PAD + HALF]],
                           axis=0).reshape(B, N, D)

  return unpad(h2), unpad(c2)
